# baseline (speedup 1.0000x reference)
import functools

import jax
import jax.numpy as jnp
from jax.experimental import pallas as pl

QB = 256
M = 16384
DPAD = 8
K = 3


def _body(q_ref, rT_ref, flow_ref, out_ref):
    q = q_ref[...]
    rT = rT_ref[...]
    flow = flow_ref[...]

    q2 = jnp.sum(q * q, axis=1, keepdims=True)
    r2 = jnp.sum(rT * rT, axis=0, keepdims=True)
    qr = jnp.dot(q, rT, preferred_element_type=jnp.float32)
    d2 = q2 - 2.0 * qr + r2
    dist = jnp.sqrt(jnp.maximum(d2, 0.0))

    iota = jax.lax.broadcasted_iota(jnp.int32, dist.shape, 1)
    inf = jnp.float32(jnp.inf)
    big = jnp.int32(2**30)

    m1 = jnp.min(dist, axis=1, keepdims=True)
    idx1 = jnp.min(jnp.where(dist == m1, iota, big), axis=1, keepdims=True)
    excl1 = iota == idx1
    m2 = jnp.min(jnp.where(excl1, inf, dist), axis=1, keepdims=True)
    idx2 = jnp.min(jnp.where((dist == m2) & ~excl1, iota, big),
                   axis=1, keepdims=True)
    excl2 = excl1 | (iota == idx2)
    m3 = jnp.min(jnp.where(excl2, inf, dist), axis=1, keepdims=True)
    idx3 = jnp.min(jnp.where((dist == m3) & ~excl2, iota, big),
                   axis=1, keepdims=True)

    w1 = 1.0 / (m1 + 1e-8)
    w2 = 1.0 / (m2 + 1e-8)
    w3 = 1.0 / (m3 + 1e-8)
    zero = jnp.float32(0.0)
    wrow = (jnp.where(iota == idx1, w1, zero)
            + jnp.where(iota == idx2, w2, zero)
            + jnp.where(iota == idx3, w3, zero))
    f = jnp.dot(wrow, flow, preferred_element_type=jnp.float32)
    out_ref[...] = f / (w1 + w2 + w3)


@jax.jit
def kernel(query_points, ref_points, ref_flow):
    n = query_points.shape[0]
    qp = jnp.zeros((n, DPAD), jnp.float32).at[:, :3].set(query_points)
    rT = jnp.zeros((DPAD, M), jnp.float32).at[:3, :].set(ref_points.T)
    fp = jnp.zeros((M, DPAD), jnp.float32).at[:, :3].set(ref_flow)

    grid = (n // QB,)
    out = pl.pallas_call(
        _body,
        grid=grid,
        in_specs=[
            pl.BlockSpec((QB, DPAD), lambda i: (i, 0)),
            pl.BlockSpec((DPAD, M), lambda i: (0, 0)),
            pl.BlockSpec((M, DPAD), lambda i: (0, 0)),
        ],
        out_specs=pl.BlockSpec((QB, DPAD), lambda i: (i, 0)),
        out_shape=jax.ShapeDtypeStruct((n, DPAD), jnp.float32),
    )(qp, rT, fp)
    return out[:, :3]
